# baseline (speedup 1.0000x reference)
import functools

import jax
import jax.numpy as jnp
from jax import lax
from jax.experimental import pallas as pl
from jax.experimental.pallas import tpu as pltpu
from jax.experimental.pallas import tpu_sc as plsc

_NC = 2
_NS = 16
_NW = _NC * _NS
_CHUNK = 128


def _sc_gather(idx, table):
  n = idx.shape[0]
  _, r = table.shape
  per_w = n // _NW
  n_chunks = per_w // _CHUNK
  idx3 = idx.reshape(_NW, n_chunks, _CHUNK)

  mesh = plsc.VectorSubcoreMesh(
      core_axis_name="c", subcore_axis_name="s",
      num_cores=_NC, num_subcores=_NS)

  @functools.partial(
      pl.kernel,
      mesh=mesh,
      out_type=jax.ShapeDtypeStruct((n, r), jnp.float32),
      scratch_types=[
          pltpu.VMEM((n_chunks, _CHUNK), jnp.int32),
          pltpu.VMEM((per_w, r), jnp.float32),
          pltpu.SemaphoreType.DMA,
      ],
      compiler_params=pltpu.CompilerParams(use_tc_tiling_on_sc=False),
  )
  def gather_kernel(table_hbm, idx_hbm, out_hbm, idx_v, rows_v, sem):
    wid = lax.axis_index("s") * _NC + lax.axis_index("c")
    pltpu.sync_copy(idx_hbm.at[wid], idx_v)
    copies = []
    for j in range(n_chunks):
      copies.append(
          pltpu.async_copy(
              table_hbm.at[idx_v.at[j]],
              rows_v.at[pl.ds(j * _CHUNK, _CHUNK)],
              sem))
    for cp in copies:
      cp.wait()
    pltpu.sync_copy(rows_v, out_hbm.at[pl.ds(wid * per_w, per_w)])

  return gather_kernel(table, idx3)


def _expand_body(z_ref, w_ref, pd_ref, seed_ref, probes_ref):
  z = z_ref[0]
  w = w_ref[...]
  x = lax.dot_general(z, w, (((1,), (1,)), ((), ())),
                      preferred_element_type=jnp.float32)
  eps = jnp.finfo(jnp.float32).eps
  ms = jnp.mean(x * x, axis=1, keepdims=True)
  seed = x * lax.rsqrt(ms + eps)
  seed_ref[0] = seed
  num_probes = pd_ref.shape[0]
  for p in range(num_probes):
    probes_ref[0, p] = seed + pd_ref[p][None, :]


def _tc_expand(z, expand_w, probe_directions, s_blk):
  b, s, r = z.shape
  d = expand_w.shape[0]
  p = probe_directions.shape[0]
  grid = (b, s // s_blk)
  return pl.pallas_call(
      _expand_body,
      grid=grid,
      in_specs=[
          pl.BlockSpec((1, s_blk, r), lambda i, j: (i, j, 0)),
          pl.BlockSpec((d, r), lambda i, j: (0, 0)),
          pl.BlockSpec((p, d), lambda i, j: (0, 0)),
      ],
      out_specs=[
          pl.BlockSpec((1, s_blk, d), lambda i, j: (i, j, 0)),
          pl.BlockSpec((1, p, s_blk, d), lambda i, j: (i, 0, j, 0)),
      ],
      out_shape=[
          jax.ShapeDtypeStruct((b, s, d), jnp.float32),
          jax.ShapeDtypeStruct((b, p, s, d), jnp.float32),
      ],
      compiler_params=pltpu.CompilerParams(
          dimension_semantics=("parallel", "parallel")),
  )(z, expand_w, probe_directions)


def kernel(token_ids, embed_low, expand_w, probe_directions):
  b, s = token_ids.shape
  _, r = embed_low.shape
  idx = token_ids.reshape(-1).astype(jnp.int32)
  z = _sc_gather(idx, embed_low)
  z = z.reshape(b, s, r)
  seed, probes = _tc_expand(z, expand_w, probe_directions, s_blk=256)
  return (seed, probes)


# speedup vs baseline: 2.1864x; 2.1864x over previous
import functools

import jax
import jax.numpy as jnp
from jax import lax
from jax.experimental import pallas as pl
from jax.experimental.pallas import tpu as pltpu
from jax.experimental.pallas import tpu_sc as plsc

_NC = 2
_NS = 16
_NW = _NC * _NS
_G = 16


def _sc_gather(idx3, table3):
  pw = idx3.shape[1] * idx3.shape[2]
  r = table3.shape[2]
  n = _NW * pw
  n_groups = pw // _G

  mesh = plsc.VectorSubcoreMesh(
      core_axis_name="c", subcore_axis_name="s",
      num_cores=_NC, num_subcores=_NS)

  @functools.partial(
      pl.kernel,
      mesh=mesh,
      out_type=jax.ShapeDtypeStruct((n, r), jnp.float32),
      scratch_types=[
          pltpu.VMEM((idx3.shape[1], idx3.shape[2]), jnp.int32),
          pltpu.VMEM((2, _G, 8, r), jnp.float32),
          pltpu.VMEM((pw, r), jnp.float32),
          pltpu.SemaphoreType.DMA,
      ],
      compiler_params=pltpu.CompilerParams(needs_layout_passes=False),
  )
  def gather_kernel(table_hbm, idx_hbm, out_hbm, idx_v, ring, compact, sem):
    wid = lax.axis_index("s") * _NC + lax.axis_index("c")
    pltpu.sync_copy(idx_hbm.at[wid], idx_v)
    lane_iota = lax.iota(jnp.int32, _G)

    def load_group(g):
      return idx_v[g >> 1, pl.ds((g & 1) * _G, _G)]

    def fire_group(g, v16):
      slot = g & 1
      for lane in range(_G):
        s = jnp.sum(jnp.where(lane_iota == lane, v16, 0))
        j = lax.shift_right_logical(s, 3)
        pltpu.async_copy(table_hbm.at[j], ring.at[slot, lane], sem)

    def extract_group(g, v16):
      slot = g & 1
      pltpu.make_async_copy(
          table_hbm.at[pl.ds(0, _G)], ring.at[slot], sem).wait()
      for lane in range(_G):
        s = jnp.sum(jnp.where(lane_iota == lane, v16, 0))
        p = lax.bitwise_and(s, 7)
        t = g * _G + lane
        for k in range(r // _G):
          compact[t, pl.ds(k * _G, _G)] = (
              ring[slot, lane, p, pl.ds(k * _G, _G)])

    v0 = load_group(jnp.int32(0))
    fire_group(jnp.int32(0), v0)

    def body(g, prev16):
      v16 = load_group(g)
      fire_group(g, v16)
      extract_group(g - 1, prev16)
      return v16

    vlast = lax.fori_loop(1, n_groups, body, v0)
    extract_group(jnp.int32(n_groups - 1), vlast)
    pltpu.sync_copy(compact, out_hbm.at[pl.ds(wid * pw, pw)])

  return gather_kernel(table3, idx3)


def _expand_body(z_ref, w_ref, pd_ref, seed_ref, probes_ref):
  z = z_ref[0]
  w = w_ref[...]
  x = lax.dot_general(z, w, (((1,), (1,)), ((), ())),
                      preferred_element_type=jnp.float32)
  eps = jnp.finfo(jnp.float32).eps
  ms = jnp.mean(x * x, axis=1, keepdims=True)
  seed = x * lax.rsqrt(ms + eps)
  seed_ref[0] = seed
  num_probes = pd_ref.shape[0]
  for p in range(num_probes):
    probes_ref[0, p] = seed + pd_ref[p][None, :]


def _tc_expand(z, expand_w, probe_directions, s_blk):
  b, s, r = z.shape
  d = expand_w.shape[0]
  p = probe_directions.shape[0]
  grid = (b, s // s_blk)
  return pl.pallas_call(
      _expand_body,
      grid=grid,
      in_specs=[
          pl.BlockSpec((1, s_blk, r), lambda i, j: (i, j, 0)),
          pl.BlockSpec((d, r), lambda i, j: (0, 0)),
          pl.BlockSpec((p, d), lambda i, j: (0, 0)),
      ],
      out_specs=[
          pl.BlockSpec((1, s_blk, d), lambda i, j: (i, j, 0)),
          pl.BlockSpec((1, p, s_blk, d), lambda i, j: (i, 0, j, 0)),
      ],
      out_shape=[
          jax.ShapeDtypeStruct((b, s, d), jnp.float32),
          jax.ShapeDtypeStruct((b, p, s, d), jnp.float32),
      ],
      compiler_params=pltpu.CompilerParams(
          dimension_semantics=("parallel", "parallel")),
  )(z, expand_w, probe_directions)


def kernel(token_ids, embed_low, expand_w, probe_directions):
  b, s = token_ids.shape
  v, r = embed_low.shape
  idx = token_ids.reshape(-1).astype(jnp.int32)
  idx3 = idx.reshape(_NW, -1, 32)
  table3 = embed_low.reshape(v // 8, 8, r)
  z = _sc_gather(idx3, table3)
  z = z.reshape(b, s, r)
  seed, probes = _tc_expand(z, expand_w, probe_directions, s_blk=256)
  return (seed, probes)
